# baseline (speedup 1.0000x reference)
import functools

import jax
import jax.numpy as jnp
from jax import lax
from jax.experimental import pallas as pl
from jax.experimental.pallas import tpu as pltpu
from jax.experimental.pallas import tpu_sc as plsc

NUM_DATA = 100000
NUM_TRUNC = 10000
D = 128
BLK = 128
FBLK = 128
SEG = 1024
NT = 16
NC = 2
PAD_DST = 1 << 30


def _make_spmm(n_edges_pad, out_rows, chunk_rows, acc_rows, n_chunks_per_sc,
               pcap):
    stripe = n_edges_pad // NT
    nseg = stripe // SEG
    cap = pcap * BLK
    assert stripe % SEG == 0
    assert chunk_rows % (NT * 32) == 0 and acc_rows % (NT * 8) == 0
    assert acc_rows >= chunk_rows + NT
    assert chunk_rows * n_chunks_per_sc * NC == out_rows
    mesh = plsc.VectorSubcoreMesh(core_axis_name="c", subcore_axis_name="s")

    @functools.partial(
        pl.kernel,
        out_type=jax.ShapeDtypeStruct((out_rows, D), jnp.float32),
        mesh=mesh,
        scratch_types=[
            pltpu.VMEM((2, 3, SEG), jnp.int32),
            pltpu.VMEM((2, FBLK), jnp.int32),
            pltpu.VMEM((2, FBLK), jnp.int32),
            pltpu.VMEM((cap + BLK,), jnp.int32),
            pltpu.VMEM((cap + BLK,), jnp.int32),
            pltpu.VMEM((cap + BLK,), jnp.float32),
            pltpu.VMEM((2, FBLK, D), jnp.float32),
            pltpu.VMEM((8, D), jnp.float32),
            pltpu.VMEM_SHARED((acc_rows, D), jnp.float32),
            pltpu.SemaphoreType.DMA,
            pltpu.SemaphoreType.DMA,
            pltpu.SemaphoreType.DMA,
            pltpu.SemaphoreType.DMA,
        ],
        compiler_params=pltpu.CompilerParams(needs_layout_passes=False),
    )
    def spmm(edges_hbm, w_hbm, table_hbm, out_hbm,
             stage, src_v, dst_v, psrc, pldst, pw, rows_v, zero_v, acc,
             sem_g, sem_s, sem_stage, sem_batch):
        cid = lax.axis_index("c")
        tid = lax.axis_index("s")
        dump_row = chunk_rows + tid
        del w_hbm

        def zrow(r, _):
            for v in range(D // 16):
                zero_v[r, pl.ds(v * 16, 16)] = jnp.zeros((16,), jnp.float32)
            return 0
        lax.fori_loop(0, 8, zrow, 0)

        def stage_idx_and_gather(b):
            par = b & 1
            for v in range(FBLK // 16):
                sl = pl.ds(v * 16, 16)
                src_v[par, sl] = psrc[pl.ds(b * FBLK + v * 16, 16)]
                dst_v[par, sl] = pldst[pl.ds(b * FBLK + v * 16, 16)]
            pltpu.async_copy(table_hbm.at[src_v.at[par]], rows_v.at[par],
                             sem_g)

        def flush_many(nb):
            @pl.when(nb > 0)
            def _():
                stage_idx_and_gather(0)

                def body(b, _):
                    par = b & 1
                    pltpu.make_async_copy(table_hbm.at[pl.ds(0, FBLK)],
                                          rows_v.at[par], sem_g).wait()

                    @pl.when(b >= 1)
                    def _():
                        pltpu.make_async_copy(rows_v.at[1 - par],
                                              acc.at[pl.ds(0, FBLK)],
                                              sem_s).wait()

                    @pl.when(b + 1 < nb)
                    def _():
                        stage_idx_and_gather(b + 1)

                    def scale(g, _):
                        w16 = pw[pl.ds(b * FBLK + g * 16, 16)]
                        for l in range(16):
                            w = w16[l]
                            j = g * 16 + l
                            for v in range(D // 16):
                                sl = pl.ds(v * 16, 16)
                                rows_v[par, j, sl] = rows_v[par, j, sl] * w
                        return 0
                    lax.fori_loop(0, FBLK // 16, scale, 0)
                    pltpu.async_copy(rows_v.at[par], acc.at[dst_v.at[par]],
                                     sem_s, add=True)
                    return 0
                lax.fori_loop(0, nb, body, 0)
                pltpu.make_async_copy(rows_v.at[0], acc.at[pl.ds(0, FBLK)],
                                      sem_s).wait()

        def chunk_body(c, _):
            cb = (cid * n_chunks_per_sc + c) * chunk_rows

            nz = acc_rows // (NT * 8)
            for z in range(nz):
                r0 = tid * (acc_rows // NT) + z * 8
                pltpu.async_copy(zero_v, acc.at[pl.ds(r0, 8)], sem_batch)
            for z in range(nz):
                pltpu.make_async_copy(zero_v, acc.at[pl.ds(0, 8)],
                                      sem_batch).wait()
            plsc.subcore_barrier()

            sbase = tid * stripe
            pltpu.async_copy(edges_hbm.at[:, pl.ds(sbase, SEG)], stage.at[0],
                             sem_stage)

            def seg_body(s, pend):
                par = s % 2
                pltpu.make_async_copy(edges_hbm.at[:, pl.ds(0, SEG)],
                                      stage.at[0], sem_stage).wait()

                @pl.when(s + 1 < nseg)
                def _():
                    pltpu.async_copy(
                        edges_hbm.at[:, pl.ds(sbase + (s + 1) * SEG, SEG)],
                        stage.at[1 - par], sem_stage)

                for blk in range(SEG // BLK):
                    for v in range(BLK // 16):
                        sl = pl.ds(blk * BLK + v * 16, 16)
                        l16 = stage[par, 1, sl] - cb
                        inr = (l16 >= 0) & (l16 < chunk_rows)
                        plsc.store_compressed(psrc.at[pl.ds(pend, 16)],
                                              stage[par, 0, sl], mask=inr)
                        plsc.store_compressed(pldst.at[pl.ds(pend, 16)],
                                              l16, mask=inr)
                        plsc.store_compressed(
                            pw.at[pl.ds(pend, 16)],
                            plsc.bitcast(stage[par, 2, sl], jnp.float32),
                            mask=inr)
                        pend = pend + plsc.all_reduce_population_count(inr)[0]

                    def overflow(p):
                        nfull = p >> 7
                        flush_many(nfull)
                        for v in range(FBLK // 16):
                            sl = pl.ds(v * 16, 16)
                            off = pl.ds(nfull * FBLK + v * 16, 16)
                            psrc[sl] = psrc[off]
                            pldst[sl] = pldst[off]
                            pw[sl] = pw[off]
                        return p & (FBLK - 1)
                    pend = lax.cond(pend >= cap - BLK, overflow,
                                    lambda p: p, pend)
                return pend
            pend = lax.fori_loop(0, nseg, seg_body, jnp.int32(0))

            for v in range(BLK // 16):
                off = pl.ds(pend + v * 16, 16)
                psrc[off] = jnp.zeros((16,), jnp.int32)
                pldst[off] = jnp.full((16,), dump_row, jnp.int32)
                pw[off] = jnp.zeros((16,), jnp.float32)
            flush_many((pend + FBLK - 1) >> 7)
            plsc.subcore_barrier()

            rpt = chunk_rows // NT
            for z in range(rpt // 32):
                r0 = tid * rpt + z * 32
                pltpu.async_copy(acc.at[pl.ds(r0, 32)],
                                 out_hbm.at[pl.ds(cb + r0, 32)], sem_batch)
            for z in range(rpt // 32):
                pltpu.make_async_copy(acc.at[pl.ds(0, 32)],
                                      out_hbm.at[pl.ds(cb, 32)],
                                      sem_batch).wait()
            plsc.subcore_barrier()
            return 0
        lax.fori_loop(0, n_chunks_per_sc, chunk_body, 0)

    return spmm


def _pack_edges(src, dst, w, n_pad):
    e = src.shape[0]
    pad = n_pad - e
    src = jnp.concatenate([src, jnp.zeros((pad,), jnp.int32)])
    dst = jnp.concatenate([dst, jnp.full((pad,), PAD_DST, jnp.int32)])
    w = jnp.concatenate([w, jnp.zeros((pad,), jnp.float32)])
    wbits = lax.bitcast_convert_type(w, jnp.int32)
    return jnp.stack([src, dst, wbits]), w


_E_PAD = 409600

_down_spmm = _make_spmm(_E_PAD, out_rows=10240, chunk_rows=5120,
                        acc_rows=5632, n_chunks_per_sc=1, pcap=112)
_up_spmm = _make_spmm(_E_PAD, out_rows=104448, chunk_rows=8704,
                      acc_rows=8832, n_chunks_per_sc=6, pcap=40)


def kernel(x, down_w, up_w, down_edge_index, up_edge_index):
    xl = x[:, -1]
    b, e, g, f = xl.shape
    x2d = xl.reshape(g, f)
    dpack, dw = _pack_edges(down_edge_index[0], down_edge_index[1],
                            down_w, _E_PAD)
    upack, uw = _pack_edges(up_edge_index[0], up_edge_index[1],
                            up_w, _E_PAD)
    down_table = _down_spmm(dpack, dw, x2d)
    up = _up_spmm(upack, uw, down_table)
    return up[:NUM_DATA].reshape(b, e, NUM_DATA, f)


# speedup vs baseline: 1.1739x; 1.1739x over previous
import functools

import jax
import jax.numpy as jnp
from jax import lax
from jax.experimental import pallas as pl
from jax.experimental.pallas import tpu as pltpu
from jax.experimental.pallas import tpu_sc as plsc

NUM_DATA = 100000
NUM_TRUNC = 10000
D = 128
BLK = 128
FBLK = 64
SEG = 1024
NT = 16
NC = 2
PAD_DST = 1 << 30


def _make_spmm(n_edges_pad, out_rows, chunk_rows, acc_rows, n_chunks_per_sc,
               pcap):
    stripe = n_edges_pad // NT
    nseg = stripe // SEG
    cap = pcap * BLK
    assert stripe % SEG == 0
    assert chunk_rows % (NT * 32) == 0 and acc_rows % (NT * 8) == 0
    assert acc_rows >= chunk_rows + NT
    assert chunk_rows * n_chunks_per_sc * NC == out_rows
    mesh = plsc.VectorSubcoreMesh(core_axis_name="c", subcore_axis_name="s")

    @functools.partial(
        pl.kernel,
        out_type=jax.ShapeDtypeStruct((out_rows, D), jnp.float32),
        mesh=mesh,
        scratch_types=[
            pltpu.VMEM((2, 3, SEG), jnp.int32),
            pltpu.VMEM((2, FBLK), jnp.int32),
            pltpu.VMEM((2, FBLK), jnp.int32),
            pltpu.VMEM((cap + BLK,), jnp.int32),
            pltpu.VMEM((cap + BLK,), jnp.int32),
            pltpu.VMEM((cap + BLK,), jnp.float32),
            pltpu.VMEM((2, FBLK, D), jnp.float32),
            pltpu.VMEM((8, D), jnp.float32),
            pltpu.VMEM_SHARED((acc_rows, D), jnp.float32),
            pltpu.SemaphoreType.DMA,
            pltpu.SemaphoreType.DMA,
            pltpu.SemaphoreType.DMA,
        ],
        compiler_params=pltpu.CompilerParams(needs_layout_passes=False),
    )
    def spmm(edges_hbm, w_hbm, table_hbm, out_hbm,
             stage, src_v, dst_v, psrc, pldst, pw, rows_v, zero_v, acc,
             sem, sem_stage, sem_batch):
        cid = lax.axis_index("c")
        tid = lax.axis_index("s")
        dump_row = chunk_rows + tid
        del w_hbm

        def zrow(r, _):
            for v in range(D // 16):
                zero_v[r, pl.ds(v * 16, 16)] = jnp.zeros((16,), jnp.float32)
            return 0
        lax.fori_loop(0, 8, zrow, 0)

        def stage_idx_and_gather(b):
            par = b & 1
            for v in range(FBLK // 16):
                sl = pl.ds(v * 16, 16)
                src_v[par, sl] = psrc[pl.ds(b * FBLK + v * 16, 16)]
                dst_v[par, sl] = pldst[pl.ds(b * FBLK + v * 16, 16)]
            pltpu.async_copy(table_hbm.at[src_v.at[par]], rows_v.at[par], sem)

        def flush_many(nb):
            @pl.when(nb > 0)
            def _():
                stage_idx_and_gather(0)

                def body(b, _):
                    par = b & 1
                    pltpu.make_async_copy(table_hbm.at[pl.ds(0, FBLK)],
                                          rows_v.at[par], sem).wait()

                    @pl.when(b + 1 < nb)
                    def _():
                        stage_idx_and_gather(b + 1)

                    def scale(g, _):
                        w16 = pw[pl.ds(b * FBLK + g * 16, 16)]
                        for l in range(16):
                            w = w16[l]
                            j = g * 16 + l
                            for v in range(D // 16):
                                sl = pl.ds(v * 16, 16)
                                rows_v[par, j, sl] = rows_v[par, j, sl] * w
                        return 0
                    lax.fori_loop(0, FBLK // 16, scale, 0)
                    pltpu.sync_copy(rows_v.at[par], acc.at[dst_v.at[par]],
                                    add=True)
                    return 0
                lax.fori_loop(0, nb, body, 0)

        def chunk_body(c, _):
            cb = (cid * n_chunks_per_sc + c) * chunk_rows

            nz = acc_rows // (NT * 8)
            for z in range(nz):
                r0 = tid * (acc_rows // NT) + z * 8
                pltpu.async_copy(zero_v, acc.at[pl.ds(r0, 8)], sem_batch)
            for z in range(nz):
                pltpu.make_async_copy(zero_v, acc.at[pl.ds(0, 8)],
                                      sem_batch).wait()
            plsc.subcore_barrier()

            sbase = tid * stripe
            pltpu.async_copy(edges_hbm.at[:, pl.ds(sbase, SEG)], stage.at[0],
                             sem_stage)

            def seg_body(s, pend):
                par = s % 2
                pltpu.make_async_copy(edges_hbm.at[:, pl.ds(0, SEG)],
                                      stage.at[0], sem_stage).wait()

                @pl.when(s + 1 < nseg)
                def _():
                    pltpu.async_copy(
                        edges_hbm.at[:, pl.ds(sbase + (s + 1) * SEG, SEG)],
                        stage.at[1 - par], sem_stage)

                for blk in range(SEG // BLK):
                    for v in range(BLK // 16):
                        sl = pl.ds(blk * BLK + v * 16, 16)
                        l16 = stage[par, 1, sl] - cb
                        inr = (l16 >= 0) & (l16 < chunk_rows)
                        plsc.store_compressed(psrc.at[pl.ds(pend, 16)],
                                              stage[par, 0, sl], mask=inr)
                        plsc.store_compressed(pldst.at[pl.ds(pend, 16)],
                                              l16, mask=inr)
                        plsc.store_compressed(
                            pw.at[pl.ds(pend, 16)],
                            plsc.bitcast(stage[par, 2, sl], jnp.float32),
                            mask=inr)
                        pend = pend + plsc.all_reduce_population_count(inr)[0]

                    def overflow(p):
                        nfull = p >> 6
                        flush_many(nfull)
                        for v in range(FBLK // 16):
                            sl = pl.ds(v * 16, 16)
                            off = pl.ds(nfull * FBLK + v * 16, 16)
                            psrc[sl] = psrc[off]
                            pldst[sl] = pldst[off]
                            pw[sl] = pw[off]
                        return p & (FBLK - 1)
                    pend = lax.cond(pend >= cap - BLK, overflow,
                                    lambda p: p, pend)
                return pend
            pend = lax.fori_loop(0, nseg, seg_body, jnp.int32(0))

            for v in range(BLK // 16):
                off = pl.ds(pend + v * 16, 16)
                psrc[off] = jnp.zeros((16,), jnp.int32)
                pldst[off] = jnp.full((16,), dump_row, jnp.int32)
                pw[off] = jnp.zeros((16,), jnp.float32)
            flush_many((pend + FBLK - 1) >> 6)
            plsc.subcore_barrier()

            rpt = chunk_rows // NT
            for z in range(rpt // 32):
                r0 = tid * rpt + z * 32
                pltpu.async_copy(acc.at[pl.ds(r0, 32)],
                                 out_hbm.at[pl.ds(cb + r0, 32)], sem_batch)
            for z in range(rpt // 32):
                pltpu.make_async_copy(acc.at[pl.ds(0, 32)],
                                      out_hbm.at[pl.ds(cb, 32)],
                                      sem_batch).wait()
            plsc.subcore_barrier()
            return 0
        lax.fori_loop(0, n_chunks_per_sc, chunk_body, 0)

    return spmm


def _pack_edges(src, dst, w, n_pad):
    e = src.shape[0]
    pad = n_pad - e
    src = jnp.concatenate([src, jnp.zeros((pad,), jnp.int32)])
    dst = jnp.concatenate([dst, jnp.full((pad,), PAD_DST, jnp.int32)])
    w = jnp.concatenate([w, jnp.zeros((pad,), jnp.float32)])
    wbits = lax.bitcast_convert_type(w, jnp.int32)
    return jnp.stack([src, dst, wbits]), w


_E_PAD = 409600

_down_spmm = _make_spmm(_E_PAD, out_rows=10240, chunk_rows=5120,
                        acc_rows=5632, n_chunks_per_sc=1, pcap=152)
_up_spmm = _make_spmm(_E_PAD, out_rows=102400, chunk_rows=10240,
                      acc_rows=10752, n_chunks_per_sc=5, pcap=48)


def kernel(x, down_w, up_w, down_edge_index, up_edge_index):
    xl = x[:, -1]
    b, e, g, f = xl.shape
    x2d = xl.reshape(g, f)
    dpack, dw = _pack_edges(down_edge_index[0], down_edge_index[1],
                            down_w, _E_PAD)
    upack, uw = _pack_edges(up_edge_index[0], up_edge_index[1],
                            up_w, _E_PAD)
    down_table = _down_spmm(dpack, dw, x2d)
    up = _up_spmm(upack, uw, down_table)
    return up[:NUM_DATA].reshape(b, e, NUM_DATA, f)
